# baseline (speedup 1.0000x reference)
import functools

import jax
import jax.numpy as jnp
from jax import lax
from jax.experimental import pallas as pl
from jax.experimental.pallas import tpu as pltpu
from jax.experimental.pallas import tpu_sc as plsc

_N, _D, _H, _O, _E = 2048, 768, 3072, 768, 8
_BN = 128
_NB = (2 * _N) // _BN + _E - 1
_P = _NB * _BN
_BH = 1536
_NH = _H // _BH
_NW = 32
_TPW = _N // _NW
_NCH = _TPW // 16


def _router_body(x_ref, wg_ref, idx_ref, wts_ref, ms_ref, bexp_ref):
    lT = lax.dot_general(wg_ref[...], x_ref[...], (((1,), (1,)), ((), ())))
    sub = lax.broadcasted_iota(jnp.int32, (_E, _N), 0)
    m1 = jnp.max(lT, axis=0, keepdims=True)
    i1 = jnp.min(jnp.where(lT >= m1, sub, _E), axis=0, keepdims=True)
    masked = jnp.where(sub == i1, -jnp.inf, lT)
    m2 = jnp.max(masked, axis=0, keepdims=True)
    i2 = jnp.min(jnp.where(masked >= m2, sub, _E), axis=0, keepdims=True)
    t = jnp.exp(m2 - m1)
    idx_ref[...] = jnp.concatenate([i1, i2], axis=0)
    wts_ref[...] = jnp.concatenate([1.0 / (1.0 + t), t / (1.0 + t)], axis=0)

    oh = (sub == i1).astype(jnp.float32) + (sub == i2).astype(jnp.float32)
    g0 = lax.broadcasted_iota(jnp.int32, (_N, 128), 0) // _TPW
    g1 = lax.broadcasted_iota(jnp.int32, (_N, 128), 1)
    grp = (g0 == g1).astype(jnp.float32)
    percnk = lax.dot_general(oh, grp, (((1,), (0,)), ((), ())))
    s0 = lax.broadcasted_iota(jnp.int32, (128, 128), 0)
    s1 = lax.broadcasted_iota(jnp.int32, (128, 128), 1)
    upper = (s0 < s1).astype(jnp.float32)
    pref = lax.dot_general(percnk, upper, (((1,), (0,)), ((), ())))
    totals = jnp.sum(percnk, axis=1, keepdims=True)
    pad_i = ((totals.astype(jnp.int32) + (_BN - 1)) // _BN) * _BN
    e0 = lax.broadcasted_iota(jnp.int32, (_E, _E), 0)
    e1 = lax.broadcasted_iota(jnp.int32, (_E, _E), 1)
    lower = (e1 < e0).astype(jnp.float32)
    base = lax.dot_general(lower, pad_i.astype(jnp.float32),
                           (((1,), (0,)), ((), ()))).astype(jnp.int32)
    ms_ref[...] = base + pref.astype(jnp.int32)
    cum_end = base + pad_i
    blk = lax.broadcasted_iota(jnp.int32, (1, 128), 1) * _BN
    cnt = jnp.sum((blk >= cum_end).astype(jnp.int32), axis=0, keepdims=True)
    bexp_row = jnp.minimum(cnt, _E - 1)
    bexp_ref[...] = jnp.where(
        lax.broadcasted_iota(jnp.int32, (_E, 128), 0) == 0, bexp_row, 0)


def _router(x, Wg):
    return pl.pallas_call(
        _router_body,
        out_shape=[
            jax.ShapeDtypeStruct((2, _N), jnp.int32),
            jax.ShapeDtypeStruct((2, _N), jnp.float32),
            jax.ShapeDtypeStruct((_E, 128), jnp.int32),
            jax.ShapeDtypeStruct((_E, 128), jnp.int32),
        ],
    )(x, Wg)


def _route_scatter_body(e01_hbm, ms_hbm, x_hbm, pos_hbm, xs_hbm,
                        ms_loc, e0v, e1v, pos0, pos1, xloc, sem):
    wid = lax.axis_index("s") * 2 + lax.axis_index("c")
    tbase = wid * _TPW
    pltpu.sync_copy(ms_hbm, ms_loc)
    pltpu.sync_copy(e01_hbm.at[0, pl.ds(tbase, _TPW)], e0v)
    pltpu.sync_copy(e01_hbm.at[1, pl.ds(tbase, _TPW)], e1v)
    li = lax.broadcasted_iota(jnp.int32, (16,), 0)
    algn = pl.multiple_of((wid // 16) * 16, 16)
    lane = wid % 16
    starts = jnp.zeros((16,), jnp.int32)
    for e in range(_E):
        vec = ms_loc[e, pl.ds(algn, 16)]
        s_e = jnp.sum(jnp.where(li == lane, vec, 0))
        starts = jnp.where(li == e, s_e, starts)

    for j in range(_NCH):
        e0c = e0v[pl.ds(j * 16, 16)]
        e1c = e1v[pl.ds(j * 16, 16)]
        pos0c = jnp.zeros((16,), jnp.int32)
        pos1c = jnp.zeros((16,), jnp.int32)
        for e in range(_E):
            m0 = e0c == e
            m1 = e1c == e
            m0i = m0.astype(jnp.int32)
            m1i = m1.astype(jnp.int32)
            c0 = plsc.cumsum(m0i)
            c1 = plsc.cumsum(m1i)
            rank0 = (c0 - m0i) + (c1 - m1i)
            rank1 = rank0 + m0i
            s = jnp.sum(jnp.where(li == e, starts, 0))
            pos0c = jnp.where(m0, s + rank0, pos0c)
            pos1c = jnp.where(m1, s + rank1, pos1c)
            pc0 = plsc.all_reduce_population_count(m0)
            pc1 = plsc.all_reduce_population_count(m1)
            starts = jnp.where(li == e, starts + pc0 + pc1, starts)
        pos0[j] = pos0c
        pos1[j] = pos1c
        pltpu.sync_copy(pos0.at[j], pos_hbm.at[0, pl.ds(tbase + j * 16, 16)])
        pltpu.sync_copy(pos1.at[j], pos_hbm.at[1, pl.ds(tbase + j * 16, 16)])
        pltpu.sync_copy(x_hbm.at[pl.ds(tbase + j * 16, 16)], xloc.at[j])

    copies = []
    for j in range(_NCH):
        copies.append(pltpu.async_copy(xloc.at[j], xs_hbm.at[pos0.at[j]], sem))
        copies.append(pltpu.async_copy(xloc.at[j], xs_hbm.at[pos1.at[j]], sem))
    for c in copies:
        c.wait()


def _mlp_body(bexp_ref, xs_ref, w1_ref, b1_ref, w2_ref, b2_ref, out_ref,
              w1b_ref, w2b_ref):
    h = pl.program_id(0)
    b = pl.program_id(1)
    e = bexp_ref[0, b]
    prev_e = bexp_ref[0, jnp.maximum(b - 1, 0)]

    @pl.when((b == 0) | (e != prev_e))
    def _recast():
        w1b_ref[...] = w1_ref[0].astype(jnp.bfloat16)
        w2b_ref[...] = w2_ref[0].astype(jnp.bfloat16)

    onehot = (lax.broadcasted_iota(jnp.int32, (_E, 1), 0) == e).astype(jnp.float32)
    b1row = lax.dot_general(onehot, b1_ref[...], (((0,), (0,)), ((), ())))
    xb = xs_ref[...].astype(jnp.bfloat16)
    hpre = lax.dot_general(xb, w1b_ref[...], (((1,), (1,)), ((), ())),
                           preferred_element_type=jnp.float32) + b1row
    hact = 0.5 * hpre * (1.0 + lax.erf(hpre * 0.7071067811865476))
    ypart = lax.dot_general(hact.astype(jnp.bfloat16), w2b_ref[...],
                            (((1,), (1,)), ((), ())),
                            preferred_element_type=jnp.float32)
    row = pl.ds(b * _BN, _BN)

    @pl.when(h == 0)
    def _init():
        b2row = lax.dot_general(onehot, b2_ref[...], (((0,), (0,)), ((), ())))
        out_ref[row, :] = ypart + b2row

    @pl.when(h > 0)
    def _acc():
        out_ref[row, :] += ypart


def _expert_mlp(bexp, xs, W1, b1, W2, b2):
    grid_spec = pltpu.PrefetchScalarGridSpec(
        num_scalar_prefetch=1,
        grid=(_NH, _NB),
        in_specs=[
            pl.BlockSpec((_BN, _D), lambda h, b, be: (b, 0)),
            pl.BlockSpec((1, _BH, _D), lambda h, b, be: (be[0, b], h, 0)),
            pl.BlockSpec((_E, _BH), lambda h, b, be: (0, h)),
            pl.BlockSpec((1, _O, _BH), lambda h, b, be: (be[0, b], 0, h)),
            pl.BlockSpec((_E, _O), lambda h, b, be: (0, 0)),
        ],
        out_specs=pl.BlockSpec((_P, _O), lambda h, b, be: (0, 0)),
        scratch_shapes=[
            pltpu.VMEM((_BH, _D), jnp.bfloat16),
            pltpu.VMEM((_O, _BH), jnp.bfloat16),
        ],
    )
    return pl.pallas_call(
        _mlp_body,
        grid_spec=grid_spec,
        out_shape=jax.ShapeDtypeStruct((_P, _O), jnp.float32),
        compiler_params=pltpu.CompilerParams(
            dimension_semantics=("arbitrary", "arbitrary"),
        ),
    )(bexp, xs, W1, b1, W2, b2)


def _combine_body(ys_hbm, pos_hbm, wts_hbm, out_hbm,
                  p0, p1, r0, r1, w0, w1, oloc, sem):
    wid = lax.axis_index("s") * 2 + lax.axis_index("c")
    tbase = wid * _TPW
    li = lax.broadcasted_iota(jnp.int32, (16,), 0)
    pltpu.sync_copy(pos_hbm.at[0, pl.ds(tbase, _TPW)], p0)
    pltpu.sync_copy(pos_hbm.at[1, pl.ds(tbase, _TPW)], p1)
    pltpu.sync_copy(wts_hbm.at[0, pl.ds(tbase, _TPW)], w0)
    pltpu.sync_copy(wts_hbm.at[1, pl.ds(tbase, _TPW)], w1)
    for half in range(2):
        hb = half * (_TPW // 2)
        cp0 = pltpu.async_copy(ys_hbm.at[p0.at[pl.ds(hb, _TPW // 2)]], r0, sem)
        cp1 = pltpu.async_copy(ys_hbm.at[p1.at[pl.ds(hb, _TPW // 2)]], r1, sem)
        cp0.wait()
        cp1.wait()
        for tt in range(2):
            w0reg = w0[pl.ds(hb + tt * 16, 16)]
            w1reg = w1[pl.ds(hb + tt * 16, 16)]
            for t in range(16):
                a = jnp.sum(jnp.where(li == t, w0reg, 0.0))
                b = jnp.sum(jnp.where(li == t, w1reg, 0.0))
                row = tt * 16 + t

                def body(c, carry, row=row, a=a, b=b):
                    for u in range(4):
                        sl = pl.ds(pl.multiple_of(c * 64 + u * 16, 16), 16)
                        oloc[row, sl] = a * r0[row, sl] + b * r1[row, sl]
                    return carry

                lax.fori_loop(0, _O // 64, body, 0)
        pltpu.sync_copy(oloc, out_hbm.at[pl.ds(tbase + hb, _TPW // 2)])


@functools.lru_cache
def _sc_kernels():
    mesh = plsc.VectorSubcoreMesh(core_axis_name="c", subcore_axis_name="s")
    route_scatter = pl.kernel(
        _route_scatter_body,
        out_type=[
            jax.ShapeDtypeStruct((2, _N), jnp.int32),
            jax.ShapeDtypeStruct((_P, _D), jnp.float32),
        ],
        mesh=mesh,
        scratch_types=[
            pltpu.VMEM((_E, 128), jnp.int32),
            pltpu.VMEM((_TPW,), jnp.int32),
            pltpu.VMEM((_TPW,), jnp.int32),
            pltpu.VMEM((_NCH, 16), jnp.int32),
            pltpu.VMEM((_NCH, 16), jnp.int32),
            pltpu.VMEM((_NCH, 16, _D), jnp.float32),
            pltpu.SemaphoreType.DMA,
        ],
        compiler_params=pltpu.CompilerParams(needs_layout_passes=False),
    )
    combine = pl.kernel(
        _combine_body,
        out_type=jax.ShapeDtypeStruct((_N, _O), jnp.float32),
        mesh=mesh,
        scratch_types=[
            pltpu.VMEM((_TPW,), jnp.int32),
            pltpu.VMEM((_TPW,), jnp.int32),
            pltpu.VMEM((_TPW // 2, _O), jnp.float32),
            pltpu.VMEM((_TPW // 2, _O), jnp.float32),
            pltpu.VMEM((_TPW,), jnp.float32),
            pltpu.VMEM((_TPW,), jnp.float32),
            pltpu.VMEM((_TPW // 2, _O), jnp.float32),
            pltpu.SemaphoreType.DMA,
        ],
        compiler_params=pltpu.CompilerParams(needs_layout_passes=False),
    )
    return route_scatter, combine


def kernel(x, Wg, W1, b1, W2, b2):
    _route_scatter, _combine = _sc_kernels()
    idx2d, wts2d, ms2d, bexp2d = _router(x, Wg)
    pos, xs = _route_scatter(idx2d, ms2d, x)
    ys = _expert_mlp(bexp2d, xs, W1, b1, W2, b2)
    return _combine(ys, pos, wts2d)


# speedup vs baseline: 1.3865x; 1.3865x over previous
import functools

import jax
import jax.numpy as jnp
from jax import lax
from jax.experimental import pallas as pl
from jax.experimental.pallas import tpu as pltpu
from jax.experimental.pallas import tpu_sc as plsc

_N, _D, _H, _O, _E = 2048, 768, 3072, 768, 8
_BN = 256
_NB = (2 * _N) // _BN + _E - 1
_P = _NB * _BN
_BH = 1536
_NH = _H // _BH
_NW = 32
_TPW = _N // _NW
_NCH = _TPW // 16


def _router_body(x_ref, wg_ref, idx_ref, wts_ref, ms_ref, bexp_ref):
    lT = lax.dot_general(wg_ref[...], x_ref[...], (((1,), (1,)), ((), ())))
    sub = lax.broadcasted_iota(jnp.int32, (_E, _N), 0)
    m1 = jnp.max(lT, axis=0, keepdims=True)
    i1 = jnp.min(jnp.where(lT >= m1, sub, _E), axis=0, keepdims=True)
    masked = jnp.where(sub == i1, -jnp.inf, lT)
    m2 = jnp.max(masked, axis=0, keepdims=True)
    i2 = jnp.min(jnp.where(masked >= m2, sub, _E), axis=0, keepdims=True)
    t = jnp.exp(m2 - m1)
    idx_ref[...] = jnp.concatenate([i1, i2], axis=0)
    wts_ref[...] = jnp.concatenate([1.0 / (1.0 + t), t / (1.0 + t)], axis=0)

    oh = (sub == i1).astype(jnp.float32) + (sub == i2).astype(jnp.float32)
    g0 = lax.broadcasted_iota(jnp.int32, (_N, 128), 0) // _TPW
    g1 = lax.broadcasted_iota(jnp.int32, (_N, 128), 1)
    grp = (g0 == g1).astype(jnp.float32)
    percnk = lax.dot_general(oh, grp, (((1,), (0,)), ((), ())))
    s0 = lax.broadcasted_iota(jnp.int32, (128, 128), 0)
    s1 = lax.broadcasted_iota(jnp.int32, (128, 128), 1)
    upper = (s0 < s1).astype(jnp.float32)
    pref = lax.dot_general(percnk, upper, (((1,), (0,)), ((), ())))
    totals = jnp.sum(percnk, axis=1, keepdims=True)
    pad_i = ((totals.astype(jnp.int32) + (_BN - 1)) // _BN) * _BN
    e0 = lax.broadcasted_iota(jnp.int32, (_E, _E), 0)
    e1 = lax.broadcasted_iota(jnp.int32, (_E, _E), 1)
    lower = (e1 < e0).astype(jnp.float32)
    base = lax.dot_general(lower, pad_i.astype(jnp.float32),
                           (((1,), (0,)), ((), ()))).astype(jnp.int32)
    ms_ref[...] = base + pref.astype(jnp.int32)
    cum_end = base + pad_i
    blk = lax.broadcasted_iota(jnp.int32, (1, 128), 1) * _BN
    cnt = jnp.sum((blk >= cum_end).astype(jnp.int32), axis=0, keepdims=True)
    bexp_row = jnp.minimum(cnt, _E - 1)
    bexp_ref[...] = jnp.where(
        lax.broadcasted_iota(jnp.int32, (_E, 128), 0) == 0, bexp_row, 0)


def _router(x, Wg):
    return pl.pallas_call(
        _router_body,
        out_shape=[
            jax.ShapeDtypeStruct((2, _N), jnp.int32),
            jax.ShapeDtypeStruct((2, _N), jnp.float32),
            jax.ShapeDtypeStruct((_E, 128), jnp.int32),
            jax.ShapeDtypeStruct((_E, 128), jnp.int32),
        ],
    )(x, Wg)


def _route_scatter_body(e01_hbm, ms_hbm, x_hbm, pos_hbm, xs_hbm,
                        ms_loc, e0v, e1v, pos0, pos1, xloc, sem, xsem):
    wid = lax.axis_index("s") * 2 + lax.axis_index("c")
    tbase = wid * _TPW
    xcopies = [
        pltpu.async_copy(x_hbm.at[pl.ds(tbase + j * 16, 16)], xloc.at[j], xsem)
        for j in range(_NCH)
    ]
    pltpu.sync_copy(ms_hbm, ms_loc)
    pltpu.sync_copy(e01_hbm.at[0, pl.ds(tbase, _TPW)], e0v)
    pltpu.sync_copy(e01_hbm.at[1, pl.ds(tbase, _TPW)], e1v)
    li = lax.broadcasted_iota(jnp.int32, (16,), 0)
    algn = pl.multiple_of((wid // 16) * 16, 16)
    lane = wid % 16
    starts = jnp.zeros((16,), jnp.int32)
    for e in range(_E):
        vec = ms_loc[e, pl.ds(algn, 16)]
        s_e = jnp.sum(jnp.where(li == lane, vec, 0))
        starts = jnp.where(li == e, s_e, starts)

    for j in range(_NCH):
        e0c = e0v[pl.ds(j * 16, 16)]
        e1c = e1v[pl.ds(j * 16, 16)]
        pos0c = jnp.zeros((16,), jnp.int32)
        pos1c = jnp.zeros((16,), jnp.int32)
        for e in range(_E):
            m0 = e0c == e
            m1 = e1c == e
            m0i = m0.astype(jnp.int32)
            m1i = m1.astype(jnp.int32)
            c0 = plsc.cumsum(m0i)
            c1 = plsc.cumsum(m1i)
            rank0 = (c0 - m0i) + (c1 - m1i)
            rank1 = rank0 + m0i
            s = jnp.sum(jnp.where(li == e, starts, 0))
            pos0c = jnp.where(m0, s + rank0, pos0c)
            pos1c = jnp.where(m1, s + rank1, pos1c)
            pc0 = plsc.all_reduce_population_count(m0)
            pc1 = plsc.all_reduce_population_count(m1)
            starts = jnp.where(li == e, starts + pc0 + pc1, starts)
        pos0[j] = pos0c
        pos1[j] = pos1c
        pltpu.sync_copy(pos0.at[j], pos_hbm.at[0, pl.ds(tbase + j * 16, 16)])
        pltpu.sync_copy(pos1.at[j], pos_hbm.at[1, pl.ds(tbase + j * 16, 16)])

    for c in xcopies:
        c.wait()
    copies = []
    for j in range(_NCH):
        copies.append(pltpu.async_copy(xloc.at[j], xs_hbm.at[pos0.at[j]], sem))
        copies.append(pltpu.async_copy(xloc.at[j], xs_hbm.at[pos1.at[j]], sem))
    for c in copies:
        c.wait()


def _mlp_body(bexp_ref, xs_ref, w1_ref, b1_ref, w2_ref, b2_ref, out_ref,
              w1b_ref, w2b_ref):
    h = pl.program_id(0)
    b = pl.program_id(1)
    e = bexp_ref[0, b]
    prev_e = bexp_ref[0, jnp.maximum(b - 1, 0)]

    @pl.when((b == 0) | (e != prev_e))
    def _recast():
        w1b_ref[...] = w1_ref[0].astype(jnp.bfloat16)
        w2b_ref[...] = w2_ref[0].astype(jnp.bfloat16)

    onehot = (lax.broadcasted_iota(jnp.int32, (_E, 1), 0) == e).astype(jnp.float32)
    b1row = lax.dot_general(onehot, b1_ref[...], (((0,), (0,)), ((), ())))
    xb = xs_ref[...].astype(jnp.bfloat16)
    hpre = lax.dot_general(xb, w1b_ref[...], (((1,), (1,)), ((), ())),
                           preferred_element_type=jnp.float32) + b1row
    hact = 0.5 * hpre * (1.0 + lax.erf(hpre * 0.7071067811865476))
    ypart = lax.dot_general(hact.astype(jnp.bfloat16), w2b_ref[...],
                            (((1,), (1,)), ((), ())),
                            preferred_element_type=jnp.float32)
    row = pl.ds(b * _BN, _BN)

    @pl.when(h == 0)
    def _init():
        b2row = lax.dot_general(onehot, b2_ref[...], (((0,), (0,)), ((), ())))
        out_ref[row, :] = ypart + b2row

    @pl.when(h > 0)
    def _acc():
        out_ref[row, :] += ypart


def _expert_mlp(bexp, xs, W1, b1, W2, b2):
    grid_spec = pltpu.PrefetchScalarGridSpec(
        num_scalar_prefetch=1,
        grid=(_NH, _NB),
        in_specs=[
            pl.BlockSpec((_BN, _D), lambda h, b, be: (b, 0)),
            pl.BlockSpec((1, _BH, _D), lambda h, b, be: (be[0, b], h, 0)),
            pl.BlockSpec((_E, _BH), lambda h, b, be: (0, h)),
            pl.BlockSpec((1, _O, _BH), lambda h, b, be: (be[0, b], 0, h)),
            pl.BlockSpec((_E, _O), lambda h, b, be: (0, 0)),
        ],
        out_specs=pl.BlockSpec((_P, _O), lambda h, b, be: (0, 0)),
        scratch_shapes=[
            pltpu.VMEM((_BH, _D), jnp.bfloat16),
            pltpu.VMEM((_O, _BH), jnp.bfloat16),
        ],
    )
    return pl.pallas_call(
        _mlp_body,
        grid_spec=grid_spec,
        out_shape=jax.ShapeDtypeStruct((_P, _O), jnp.float32),
        compiler_params=pltpu.CompilerParams(
            dimension_semantics=("arbitrary", "arbitrary"),
        ),
    )(bexp, xs, W1, b1, W2, b2)


def _combine_body(ys_hbm, pos_hbm, wts_hbm, out_hbm,
                  p0, p1, ra0, ra1, rb0, rb1, w0, w1, oloc, sem, semb):
    wid = lax.axis_index("s") * 2 + lax.axis_index("c")
    tbase = wid * _TPW
    li = lax.broadcasted_iota(jnp.int32, (16,), 0)
    pltpu.sync_copy(pos_hbm.at[0, pl.ds(tbase, _TPW)], p0)
    pltpu.sync_copy(pos_hbm.at[1, pl.ds(tbase, _TPW)], p1)
    pltpu.sync_copy(wts_hbm.at[0, pl.ds(tbase, _TPW)], w0)
    pltpu.sync_copy(wts_hbm.at[1, pl.ds(tbase, _TPW)], w1)
    hw = _TPW // 2
    gathers = []
    for half, (g0, g1, gs) in enumerate(((ra0, ra1, sem), (rb0, rb1, semb))):
        hb = half * hw
        gathers.append(pltpu.async_copy(ys_hbm.at[p0.at[pl.ds(hb, hw)]], g0, gs))
        gathers.append(pltpu.async_copy(ys_hbm.at[p1.at[pl.ds(hb, hw)]], g1, gs))
    for half, (r0, r1) in enumerate(((ra0, ra1), (rb0, rb1))):
        hb = half * hw
        gathers[2 * half].wait()
        gathers[2 * half + 1].wait()
        for tt in range(2):
            w0reg = w0[pl.ds(hb + tt * 16, 16)]
            w1reg = w1[pl.ds(hb + tt * 16, 16)]
            for t in range(16):
                a = jnp.sum(jnp.where(li == t, w0reg, 0.0))
                b = jnp.sum(jnp.where(li == t, w1reg, 0.0))
                row = tt * 16 + t

                def body(c, carry, row=row, a=a, b=b):
                    for u in range(4):
                        sl = pl.ds(pl.multiple_of(c * 64 + u * 16, 16), 16)
                        oloc[row, sl] = a * r0[row, sl] + b * r1[row, sl]
                    return carry

                lax.fori_loop(0, _O // 64, body, 0)
        pltpu.sync_copy(oloc, out_hbm.at[pl.ds(tbase + hb, hw)])


@functools.lru_cache
def _sc_kernels():
    mesh = plsc.VectorSubcoreMesh(core_axis_name="c", subcore_axis_name="s")
    route_scatter = pl.kernel(
        _route_scatter_body,
        out_type=[
            jax.ShapeDtypeStruct((2, _N), jnp.int32),
            jax.ShapeDtypeStruct((_P, _D), jnp.float32),
        ],
        mesh=mesh,
        scratch_types=[
            pltpu.VMEM((_E, 128), jnp.int32),
            pltpu.VMEM((_TPW,), jnp.int32),
            pltpu.VMEM((_TPW,), jnp.int32),
            pltpu.VMEM((_NCH, 16), jnp.int32),
            pltpu.VMEM((_NCH, 16), jnp.int32),
            pltpu.VMEM((_NCH, 16, _D), jnp.float32),
            pltpu.SemaphoreType.DMA,
            pltpu.SemaphoreType.DMA,
        ],
        compiler_params=pltpu.CompilerParams(needs_layout_passes=False),
    )
    combine = pl.kernel(
        _combine_body,
        out_type=jax.ShapeDtypeStruct((_N, _O), jnp.float32),
        mesh=mesh,
        scratch_types=[
            pltpu.VMEM((_TPW,), jnp.int32),
            pltpu.VMEM((_TPW,), jnp.int32),
            pltpu.VMEM((_TPW // 2, _O), jnp.float32),
            pltpu.VMEM((_TPW // 2, _O), jnp.float32),
            pltpu.VMEM((_TPW // 2, _O), jnp.float32),
            pltpu.VMEM((_TPW // 2, _O), jnp.float32),
            pltpu.VMEM((_TPW,), jnp.float32),
            pltpu.VMEM((_TPW,), jnp.float32),
            pltpu.VMEM((_TPW // 2, _O), jnp.float32),
            pltpu.SemaphoreType.DMA,
            pltpu.SemaphoreType.DMA,
        ],
        compiler_params=pltpu.CompilerParams(needs_layout_passes=False),
    )
    return route_scatter, combine


def kernel(x, Wg, W1, b1, W2, b2):
    _route_scatter, _combine = _sc_kernels()
    idx2d, wts2d, ms2d, bexp2d = _router(x, Wg)
    pos, xs = _route_scatter(idx2d, ms2d, x)
    ys = _expert_mlp(bexp2d, xs, W1, b1, W2, b2)
    return _combine(ys, pos, wts2d)


# speedup vs baseline: 1.4636x; 1.0556x over previous
import functools

import jax
import jax.numpy as jnp
from jax import lax
from jax.experimental import pallas as pl
from jax.experimental.pallas import tpu as pltpu
from jax.experimental.pallas import tpu_sc as plsc

_N, _D, _H, _O, _E = 2048, 768, 3072, 768, 8
_BN = 384
_NB = (2 * _N) // _BN + _E - 1
_P = _NB * _BN
_BH = 1536
_NH = _H // _BH
_NW = 32
_TPW = _N // _NW
_NCH = _TPW // 16


def _router_body(x_ref, wg_ref, idx_ref, wts_ref, ms_ref, bexp_ref):
    lT = lax.dot_general(wg_ref[...], x_ref[...], (((1,), (1,)), ((), ())))
    sub = lax.broadcasted_iota(jnp.int32, (_E, _N), 0)
    m1 = jnp.max(lT, axis=0, keepdims=True)
    i1 = jnp.min(jnp.where(lT >= m1, sub, _E), axis=0, keepdims=True)
    masked = jnp.where(sub == i1, -jnp.inf, lT)
    m2 = jnp.max(masked, axis=0, keepdims=True)
    i2 = jnp.min(jnp.where(masked >= m2, sub, _E), axis=0, keepdims=True)
    t = jnp.exp(m2 - m1)
    idx_ref[...] = jnp.concatenate([i1, i2], axis=0)
    wts_ref[...] = jnp.concatenate([1.0 / (1.0 + t), t / (1.0 + t)], axis=0)

    oh = (sub == i1).astype(jnp.float32) + (sub == i2).astype(jnp.float32)
    g0 = lax.broadcasted_iota(jnp.int32, (_N, 128), 0) // _TPW
    g1 = lax.broadcasted_iota(jnp.int32, (_N, 128), 1)
    grp = (g0 == g1).astype(jnp.float32)
    percnk = lax.dot_general(oh, grp, (((1,), (0,)), ((), ())))
    s0 = lax.broadcasted_iota(jnp.int32, (128, 128), 0)
    s1 = lax.broadcasted_iota(jnp.int32, (128, 128), 1)
    upper = (s0 < s1).astype(jnp.float32)
    pref = lax.dot_general(percnk, upper, (((1,), (0,)), ((), ())))
    totals = jnp.sum(percnk, axis=1, keepdims=True)
    pad_i = ((totals.astype(jnp.int32) + (_BN - 1)) // _BN) * _BN
    e0 = lax.broadcasted_iota(jnp.int32, (_E, _E), 0)
    e1 = lax.broadcasted_iota(jnp.int32, (_E, _E), 1)
    lower = (e1 < e0).astype(jnp.float32)
    base = lax.dot_general(lower, pad_i.astype(jnp.float32),
                           (((1,), (0,)), ((), ()))).astype(jnp.int32)
    ms_ref[...] = base + pref.astype(jnp.int32)
    cum_end = base + pad_i
    blk = lax.broadcasted_iota(jnp.int32, (1, 128), 1) * _BN
    cnt = jnp.sum((blk >= cum_end).astype(jnp.int32), axis=0, keepdims=True)
    bexp_row = jnp.minimum(cnt, _E - 1)
    bexp_ref[...] = jnp.where(
        lax.broadcasted_iota(jnp.int32, (_E, 128), 0) == 0, bexp_row, 0)


def _router(x, Wg):
    return pl.pallas_call(
        _router_body,
        out_shape=[
            jax.ShapeDtypeStruct((2, _N), jnp.int32),
            jax.ShapeDtypeStruct((2, _N), jnp.float32),
            jax.ShapeDtypeStruct((_E, 128), jnp.int32),
            jax.ShapeDtypeStruct((_E, 128), jnp.int32),
        ],
    )(x, Wg)


def _route_scatter_body(e01_hbm, ms_hbm, x_hbm, pos_hbm, xs_hbm,
                        ms_loc, e0v, e1v, pos0, pos1, xloc, sem, xsem):
    wid = lax.axis_index("s") * 2 + lax.axis_index("c")
    tbase = wid * _TPW
    xcopies = [
        pltpu.async_copy(x_hbm.at[pl.ds(tbase + j * 16, 16)], xloc.at[j], xsem)
        for j in range(_NCH)
    ]
    pltpu.sync_copy(ms_hbm, ms_loc)
    pltpu.sync_copy(e01_hbm.at[0, pl.ds(tbase, _TPW)], e0v)
    pltpu.sync_copy(e01_hbm.at[1, pl.ds(tbase, _TPW)], e1v)
    li = lax.broadcasted_iota(jnp.int32, (16,), 0)
    algn = pl.multiple_of((wid // 16) * 16, 16)
    lane = wid % 16
    starts = jnp.zeros((16,), jnp.int32)
    for e in range(_E):
        vec = ms_loc[e, pl.ds(algn, 16)]
        s_e = jnp.sum(jnp.where(li == lane, vec, 0))
        starts = jnp.where(li == e, s_e, starts)

    for j in range(_NCH):
        e0c = e0v[pl.ds(j * 16, 16)]
        e1c = e1v[pl.ds(j * 16, 16)]
        pos0c = jnp.zeros((16,), jnp.int32)
        pos1c = jnp.zeros((16,), jnp.int32)
        for e in range(_E):
            m0 = e0c == e
            m1 = e1c == e
            m0i = m0.astype(jnp.int32)
            m1i = m1.astype(jnp.int32)
            c0 = plsc.cumsum(m0i)
            c1 = plsc.cumsum(m1i)
            rank0 = (c0 - m0i) + (c1 - m1i)
            rank1 = rank0 + m0i
            s = jnp.sum(jnp.where(li == e, starts, 0))
            pos0c = jnp.where(m0, s + rank0, pos0c)
            pos1c = jnp.where(m1, s + rank1, pos1c)
            pc0 = plsc.all_reduce_population_count(m0)
            pc1 = plsc.all_reduce_population_count(m1)
            starts = jnp.where(li == e, starts + pc0 + pc1, starts)
        pos0[j] = pos0c
        pos1[j] = pos1c
        pltpu.sync_copy(pos0.at[j], pos_hbm.at[0, pl.ds(tbase + j * 16, 16)])
        pltpu.sync_copy(pos1.at[j], pos_hbm.at[1, pl.ds(tbase + j * 16, 16)])

    for c in xcopies:
        c.wait()
    copies = []
    for j in range(_NCH):
        copies.append(pltpu.async_copy(xloc.at[j], xs_hbm.at[pos0.at[j]], sem))
        copies.append(pltpu.async_copy(xloc.at[j], xs_hbm.at[pos1.at[j]], sem))
    for c in copies:
        c.wait()


def _mlp_body(bexp_ref, xs_ref, w1_ref, b1_ref, w2_ref, b2_ref, out_ref,
              w1b_ref, w2b_ref):
    h = pl.program_id(0)
    b = pl.program_id(1)
    e = bexp_ref[0, b]
    prev_e = bexp_ref[0, jnp.maximum(b - 1, 0)]

    @pl.when((b == 0) | (e != prev_e))
    def _recast():
        w1b_ref[...] = w1_ref[0].astype(jnp.bfloat16)
        w2b_ref[...] = w2_ref[0].astype(jnp.bfloat16)

    onehot = (lax.broadcasted_iota(jnp.int32, (_E, 1), 0) == e).astype(jnp.float32)
    b1row = lax.dot_general(onehot, b1_ref[...], (((0,), (0,)), ((), ())))
    xb = xs_ref[...].astype(jnp.bfloat16)
    hpre = lax.dot_general(xb, w1b_ref[...], (((1,), (1,)), ((), ())),
                           preferred_element_type=jnp.float32) + b1row
    hact = 0.5 * hpre * (1.0 + lax.erf(hpre * 0.7071067811865476))
    ypart = lax.dot_general(hact.astype(jnp.bfloat16), w2b_ref[...],
                            (((1,), (1,)), ((), ())),
                            preferred_element_type=jnp.float32)
    row = pl.ds(b * _BN, _BN)

    @pl.when(h == 0)
    def _init():
        b2row = lax.dot_general(onehot, b2_ref[...], (((0,), (0,)), ((), ())))
        out_ref[row, :] = ypart + b2row

    @pl.when(h > 0)
    def _acc():
        out_ref[row, :] += ypart


def _expert_mlp(bexp, xs, W1, b1, W2, b2):
    grid_spec = pltpu.PrefetchScalarGridSpec(
        num_scalar_prefetch=1,
        grid=(_NH, _NB),
        in_specs=[
            pl.BlockSpec((_BN, _D), lambda h, b, be: (b, 0)),
            pl.BlockSpec((1, _BH, _D), lambda h, b, be: (be[0, b], h, 0)),
            pl.BlockSpec((_E, _BH), lambda h, b, be: (0, h)),
            pl.BlockSpec((1, _O, _BH), lambda h, b, be: (be[0, b], 0, h)),
            pl.BlockSpec((_E, _O), lambda h, b, be: (0, 0)),
        ],
        out_specs=pl.BlockSpec((_P, _O), lambda h, b, be: (0, 0)),
        scratch_shapes=[
            pltpu.VMEM((_BH, _D), jnp.bfloat16),
            pltpu.VMEM((_O, _BH), jnp.bfloat16),
        ],
    )
    return pl.pallas_call(
        _mlp_body,
        grid_spec=grid_spec,
        out_shape=jax.ShapeDtypeStruct((_P, _O), jnp.float32),
        compiler_params=pltpu.CompilerParams(
            dimension_semantics=("arbitrary", "arbitrary"),
        ),
    )(bexp, xs, W1, b1, W2, b2)


def _combine_body(ys_hbm, pos_hbm, wts_hbm, out_hbm,
                  p0, p1, ra0, ra1, rb0, rb1, w0, w1, oloc, sem, semb):
    wid = lax.axis_index("s") * 2 + lax.axis_index("c")
    tbase = wid * _TPW
    li = lax.broadcasted_iota(jnp.int32, (16,), 0)
    pltpu.sync_copy(pos_hbm.at[0, pl.ds(tbase, _TPW)], p0)
    pltpu.sync_copy(pos_hbm.at[1, pl.ds(tbase, _TPW)], p1)
    pltpu.sync_copy(wts_hbm.at[0, pl.ds(tbase, _TPW)], w0)
    pltpu.sync_copy(wts_hbm.at[1, pl.ds(tbase, _TPW)], w1)
    hw = _TPW // 2
    gathers = []
    for half, (g0, g1, gs) in enumerate(((ra0, ra1, sem), (rb0, rb1, semb))):
        hb = half * hw
        gathers.append(pltpu.async_copy(ys_hbm.at[p0.at[pl.ds(hb, hw)]], g0, gs))
        gathers.append(pltpu.async_copy(ys_hbm.at[p1.at[pl.ds(hb, hw)]], g1, gs))
    for half, (r0, r1) in enumerate(((ra0, ra1), (rb0, rb1))):
        hb = half * hw
        gathers[2 * half].wait()
        gathers[2 * half + 1].wait()
        for tt in range(2):
            w0reg = w0[pl.ds(hb + tt * 16, 16)]
            w1reg = w1[pl.ds(hb + tt * 16, 16)]
            for t in range(16):
                a = jnp.sum(jnp.where(li == t, w0reg, 0.0))
                b = jnp.sum(jnp.where(li == t, w1reg, 0.0))
                row = tt * 16 + t

                def body(c, carry, row=row, a=a, b=b):
                    for u in range(4):
                        sl = pl.ds(pl.multiple_of(c * 64 + u * 16, 16), 16)
                        oloc[row, sl] = a * r0[row, sl] + b * r1[row, sl]
                    return carry

                lax.fori_loop(0, _O // 64, body, 0)
        pltpu.sync_copy(oloc, out_hbm.at[pl.ds(tbase + hb, hw)])


@functools.lru_cache
def _sc_kernels():
    mesh = plsc.VectorSubcoreMesh(core_axis_name="c", subcore_axis_name="s")
    route_scatter = pl.kernel(
        _route_scatter_body,
        out_type=[
            jax.ShapeDtypeStruct((2, _N), jnp.int32),
            jax.ShapeDtypeStruct((_P, _D), jnp.float32),
        ],
        mesh=mesh,
        scratch_types=[
            pltpu.VMEM((_E, 128), jnp.int32),
            pltpu.VMEM((_TPW,), jnp.int32),
            pltpu.VMEM((_TPW,), jnp.int32),
            pltpu.VMEM((_NCH, 16), jnp.int32),
            pltpu.VMEM((_NCH, 16), jnp.int32),
            pltpu.VMEM((_NCH, 16, _D), jnp.float32),
            pltpu.SemaphoreType.DMA,
            pltpu.SemaphoreType.DMA,
        ],
        compiler_params=pltpu.CompilerParams(needs_layout_passes=False),
    )
    combine = pl.kernel(
        _combine_body,
        out_type=jax.ShapeDtypeStruct((_N, _O), jnp.float32),
        mesh=mesh,
        scratch_types=[
            pltpu.VMEM((_TPW,), jnp.int32),
            pltpu.VMEM((_TPW,), jnp.int32),
            pltpu.VMEM((_TPW // 2, _O), jnp.float32),
            pltpu.VMEM((_TPW // 2, _O), jnp.float32),
            pltpu.VMEM((_TPW // 2, _O), jnp.float32),
            pltpu.VMEM((_TPW // 2, _O), jnp.float32),
            pltpu.VMEM((_TPW,), jnp.float32),
            pltpu.VMEM((_TPW,), jnp.float32),
            pltpu.VMEM((_TPW // 2, _O), jnp.float32),
            pltpu.SemaphoreType.DMA,
            pltpu.SemaphoreType.DMA,
        ],
        compiler_params=pltpu.CompilerParams(needs_layout_passes=False),
    )
    return route_scatter, combine


def kernel(x, Wg, W1, b1, W2, b2):
    _route_scatter, _combine = _sc_kernels()
    idx2d, wts2d, ms2d, bexp2d = _router(x, Wg)
    pos, xs = _route_scatter(idx2d, ms2d, x)
    ys = _expert_mlp(bexp2d, xs, W1, b1, W2, b2)
    return _combine(ys, pos, wts2d)
